# baseline (speedup 1.0000x reference)
import jax
import jax.numpy as jnp
from jax.experimental import pallas as pl
from jax.experimental.pallas import tpu as pltpu

VOCAB = 8192
ROWS_PER_STEP = 8


def _body(x_sref, t_sref, *refs):
    R = ROWS_PER_STEP
    in_refs = refs[:R]
    out_ref = refs[R]
    loss_ref = refs[R + 1]
    acc_ref = refs[R + 2]
    i = pl.program_id(0)
    nsteps = pl.num_programs(0)

    block = jnp.concatenate(
        [r[...].reshape(1, VOCAB) for r in in_refs], axis=0)
    out_ref[...] = block

    m = jnp.max(block, axis=-1, keepdims=True)
    e = jnp.exp(block - m)
    s = jnp.sum(e, axis=-1, keepdims=True)
    lse = m + jnp.log(s)

    tvec = jnp.stack([t_sref[i * R + j] for j in range(R)]).reshape(R, 1)
    col = jax.lax.broadcasted_iota(jnp.int32, (R, VOCAB), 1)
    tl = jnp.sum(jnp.where(col == tvec, block, 0.0), axis=-1, keepdims=True)

    step_nll = jnp.sum(lse - tl)
    prev = jnp.where(i == 0, 0.0, acc_ref[0])
    total = prev + step_nll
    acc_ref[0] = total

    @pl.when(i == nsteps - 1)
    def _():
        loss_ref[0, 0] = total / (nsteps * R)


def kernel(x, targets, embed_weight):
    B, T = x.shape
    N = B * T
    R = ROWS_PER_STEP
    x_flat = x.reshape(N).astype(jnp.int32)
    t_flat = targets.reshape(N).astype(jnp.int32)

    def make_in_spec(j):
        return pl.BlockSpec(
            (1, 1, VOCAB),
            lambda i, xs, ts, j=j: (xs[i * R + j], 0, 0),
        )

    grid_spec = pltpu.PrefetchScalarGridSpec(
        num_scalar_prefetch=2,
        grid=(N // R,),
        in_specs=[make_in_spec(j) for j in range(R)],
        out_specs=[
            pl.BlockSpec((R, VOCAB), lambda i, xs, ts: (i, 0)),
            pl.BlockSpec(memory_space=pltpu.SMEM),
        ],
        scratch_shapes=[pltpu.SMEM((1,), jnp.float32)],
    )

    logits, loss = pl.pallas_call(
        _body,
        grid_spec=grid_spec,
        out_shape=[
            jax.ShapeDtypeStruct((N, VOCAB), jnp.float32),
            jax.ShapeDtypeStruct((1, 1), jnp.float32),
        ],
    )(x_flat, t_flat, *([embed_weight.reshape(VOCAB, 1, VOCAB)] * R))

    return (logits, loss[0, 0])


# speedup vs baseline: 1.0158x; 1.0158x over previous
import jax
import jax.numpy as jnp
from jax.experimental import pallas as pl
from jax.experimental.pallas import tpu as pltpu

VOCAB = 8192
SUB = 8
LANE = VOCAB // SUB
ROWS_PER_STEP = 8


def _body(x_sref, t_sref, *refs):
    R = ROWS_PER_STEP
    in_refs = refs[:R]
    out_ref = refs[R]
    loss_ref = refs[R + 1]
    acc_ref = refs[R + 2]
    i = pl.program_id(0)
    nsteps = pl.num_programs(0)

    block = jnp.concatenate([r[...] for r in in_refs], axis=0)
    out_ref[...] = block

    m = jnp.max(block, axis=(1, 2), keepdims=True)
    e = jnp.exp(block - m)
    s = jnp.sum(e, axis=(1, 2), keepdims=True)
    lse = (m + jnp.log(s)).reshape(R)

    tvec = jnp.stack([t_sref[i * R + j] for j in range(R)])
    sub_iota = jax.lax.broadcasted_iota(jnp.int32, (R, SUB, LANE), 1)
    lane_iota = jax.lax.broadcasted_iota(jnp.int32, (R, SUB, LANE), 2)
    tcol = tvec[:, None, None]
    mask = (sub_iota == tcol // LANE) & (lane_iota == tcol % LANE)
    tl = jnp.sum(jnp.where(mask, block, 0.0), axis=(1, 2))

    step_nll = jnp.sum(lse - tl)
    prev = jnp.where(i == 0, 0.0, acc_ref[0])
    total = prev + step_nll
    acc_ref[0] = total

    @pl.when(i == nsteps - 1)
    def _():
        loss_ref[0, 0] = total / (nsteps * R)


def kernel(x, targets, embed_weight):
    B, T = x.shape
    N = B * T
    R = ROWS_PER_STEP
    x_flat = x.reshape(N).astype(jnp.int32)
    t_flat = targets.reshape(N).astype(jnp.int32)

    def make_in_spec(j):
        return pl.BlockSpec(
            (1, SUB, LANE),
            lambda i, xs, ts, j=j: (xs[i * R + j], 0, 0),
        )

    grid_spec = pltpu.PrefetchScalarGridSpec(
        num_scalar_prefetch=2,
        grid=(N // R,),
        in_specs=[make_in_spec(j) for j in range(R)],
        out_specs=[
            pl.BlockSpec((R, SUB, LANE), lambda i, xs, ts: (i, 0, 0)),
            pl.BlockSpec(memory_space=pltpu.SMEM),
        ],
        scratch_shapes=[pltpu.SMEM((1,), jnp.float32)],
    )

    logits3, loss = pl.pallas_call(
        _body,
        grid_spec=grid_spec,
        out_shape=[
            jax.ShapeDtypeStruct((N, SUB, LANE), jnp.float32),
            jax.ShapeDtypeStruct((1, 1), jnp.float32),
        ],
    )(x_flat, t_flat, *([embed_weight.reshape(VOCAB, SUB, LANE)] * R))

    return (logits3.reshape(N, VOCAB), loss[0, 0])


# speedup vs baseline: 2.1322x; 2.0991x over previous
import jax
import jax.numpy as jnp
from jax.experimental import pallas as pl
from jax.experimental.pallas import tpu as pltpu

VOCAB = 8192
CH = 16


def _body(x_sref, t_sref, table, out, loss_ref, buf, in_sem, out_sem):
    N = x_sref.shape[0]
    nchunks = N // CH

    def issue_gather(c, slot):
        base = c * CH
        for j in range(CH):
            pltpu.make_async_copy(
                table.at[x_sref[base + j]],
                buf.at[slot, j],
                in_sem.at[slot],
            ).start()

    def wait_gather(slot):
        for j in range(CH):
            pltpu.make_async_copy(
                table.at[0],
                buf.at[slot, j],
                in_sem.at[slot],
            ).wait()

    def issue_out(c, slot):
        pltpu.make_async_copy(
            buf.at[slot],
            out.at[pl.ds(c * CH, CH)],
            out_sem.at[slot],
        ).start()

    def wait_out(slot):
        pltpu.make_async_copy(
            buf.at[slot],
            out.at[pl.ds(0, CH)],
            out_sem.at[slot],
        ).wait()

    issue_gather(0, 0)

    def step(c, acc):
        slot = jax.lax.rem(c, 2)
        nslot = 1 - slot

        @pl.when(c + 1 < nchunks)
        def _():
            @pl.when(c >= 1)
            def _():
                wait_out(nslot)

            issue_gather(c + 1, nslot)

        wait_gather(slot)
        block = buf[slot]
        m = jnp.max(block, axis=1, keepdims=True)
        e = jnp.exp(block - m)
        s = jnp.sum(e, axis=1, keepdims=True)
        lse = (m + jnp.log(s)).reshape(CH)

        base = c * CH
        tvec = jnp.stack([t_sref[base + j] for j in range(CH)])
        lane = jax.lax.broadcasted_iota(jnp.int32, (CH, VOCAB), 1)
        tl = jnp.sum(jnp.where(lane == tvec[:, None], block, 0.0), axis=1)

        issue_out(c, slot)
        return acc + jnp.sum(lse - tl)

    total = jax.lax.fori_loop(0, nchunks, step, jnp.float32(0.0))
    wait_out(0)
    wait_out(1)
    loss_ref[0, 0] = total / N


def kernel(x, targets, embed_weight):
    B, T = x.shape
    N = B * T
    x_flat = x.reshape(N).astype(jnp.int32)
    t_flat = targets.reshape(N).astype(jnp.int32)

    grid_spec = pltpu.PrefetchScalarGridSpec(
        num_scalar_prefetch=2,
        grid=(1,),
        in_specs=[pl.BlockSpec(memory_space=pl.ANY)],
        out_specs=[
            pl.BlockSpec(memory_space=pl.ANY),
            pl.BlockSpec(memory_space=pltpu.SMEM),
        ],
        scratch_shapes=[
            pltpu.VMEM((2, CH, VOCAB), jnp.float32),
            pltpu.SemaphoreType.DMA((2,)),
            pltpu.SemaphoreType.DMA((2,)),
        ],
    )

    logits, loss = pl.pallas_call(
        _body,
        grid_spec=grid_spec,
        out_shape=[
            jax.ShapeDtypeStruct((N, VOCAB), jnp.float32),
            jax.ShapeDtypeStruct((1, 1), jnp.float32),
        ],
    )(x_flat, t_flat, embed_weight)

    return (logits, loss[0, 0])


# speedup vs baseline: 2.5883x; 1.2139x over previous
import jax
import jax.numpy as jnp
from jax.experimental import pallas as pl
from jax.experimental.pallas import tpu as pltpu

VOCAB = 8192
CH = 16
NBUF = 4


def _body(x_sref, t_sref, table, out, loss_ref, buf, in_sem, out_sem):
    N = x_sref.shape[0]
    nchunks = N // CH

    def issue_gather(c, slot):
        base = c * CH
        for j in range(CH):
            pltpu.make_async_copy(
                table.at[x_sref[base + j]],
                buf.at[slot, j],
                in_sem.at[slot],
            ).start()

    def wait_gather(slot):
        for j in range(CH):
            pltpu.make_async_copy(
                table.at[0],
                buf.at[slot, j],
                in_sem.at[slot],
            ).wait()

    def issue_out(c, slot):
        pltpu.make_async_copy(
            buf.at[slot],
            out.at[pl.ds(c * CH, CH)],
            out_sem.at[slot],
        ).start()

    def wait_out(slot):
        pltpu.make_async_copy(
            buf.at[slot],
            out.at[pl.ds(0, CH)],
            out_sem.at[slot],
        ).wait()

    for c0 in range(NBUF - 1):
        issue_gather(c0, c0)

    def step(c, acc):
        slot = jax.lax.rem(c, NBUF)
        ahead_slot = jax.lax.rem(c + NBUF - 1, NBUF)

        @pl.when(c + NBUF - 1 < nchunks)
        def _():
            @pl.when(c >= 1)
            def _():
                wait_out(ahead_slot)

            issue_gather(c + NBUF - 1, ahead_slot)

        wait_gather(slot)
        block = buf[slot]
        m = jnp.max(block, axis=1, keepdims=True)
        e = jnp.exp(block - m)
        s = jnp.sum(e, axis=1, keepdims=True)
        lse = (m + jnp.log(s)).reshape(CH)

        base = c * CH
        tvec = jnp.stack([t_sref[base + j] for j in range(CH)])
        lane = jax.lax.broadcasted_iota(jnp.int32, (CH, VOCAB), 1)
        tl = jnp.sum(jnp.where(lane == tvec[:, None], block, 0.0), axis=1)

        issue_out(c, slot)
        return acc + jnp.sum(lse - tl)

    total = jax.lax.fori_loop(0, nchunks, step, jnp.float32(0.0))
    for s0 in range(NBUF):
        wait_out(s0)
    loss_ref[0, 0] = total / N


def kernel(x, targets, embed_weight):
    B, T = x.shape
    N = B * T
    x_flat = x.reshape(N).astype(jnp.int32)
    t_flat = targets.reshape(N).astype(jnp.int32)

    grid_spec = pltpu.PrefetchScalarGridSpec(
        num_scalar_prefetch=2,
        grid=(1,),
        in_specs=[pl.BlockSpec(memory_space=pl.ANY)],
        out_specs=[
            pl.BlockSpec(memory_space=pl.ANY),
            pl.BlockSpec(memory_space=pltpu.SMEM),
        ],
        scratch_shapes=[
            pltpu.VMEM((NBUF, CH, VOCAB), jnp.float32),
            pltpu.SemaphoreType.DMA((NBUF,)),
            pltpu.SemaphoreType.DMA((NBUF,)),
        ],
    )

    logits, loss = pl.pallas_call(
        _body,
        grid_spec=grid_spec,
        out_shape=[
            jax.ShapeDtypeStruct((N, VOCAB), jnp.float32),
            jax.ShapeDtypeStruct((1, 1), jnp.float32),
        ],
    )(x_flat, t_flat, embed_weight)

    return (logits, loss[0, 0])


# speedup vs baseline: 2.5973x; 1.0035x over previous
import jax
import jax.numpy as jnp
from jax.experimental import pallas as pl
from jax.experimental.pallas import tpu as pltpu

VOCAB = 8192
CH = 16
NBUF = 8


def _body(x_sref, t_sref, table, out, loss_ref, buf, in_sem, out_sem):
    N = x_sref.shape[0]
    nchunks = N // CH

    def issue_gather(c, slot):
        base = c * CH
        for j in range(CH):
            pltpu.make_async_copy(
                table.at[x_sref[base + j]],
                buf.at[slot, j],
                in_sem.at[slot],
            ).start()

    def wait_gather(slot):
        for j in range(CH):
            pltpu.make_async_copy(
                table.at[0],
                buf.at[slot, j],
                in_sem.at[slot],
            ).wait()

    def issue_out(c, slot):
        pltpu.make_async_copy(
            buf.at[slot],
            out.at[pl.ds(c * CH, CH)],
            out_sem.at[slot],
        ).start()

    def wait_out(slot):
        pltpu.make_async_copy(
            buf.at[slot],
            out.at[pl.ds(0, CH)],
            out_sem.at[slot],
        ).wait()

    for c0 in range(NBUF - 1):
        issue_gather(c0, c0)

    def step(c, acc):
        slot = jax.lax.rem(c, NBUF)
        ahead_slot = jax.lax.rem(c + NBUF - 1, NBUF)

        @pl.when(c + NBUF - 1 < nchunks)
        def _():
            @pl.when(c >= 1)
            def _():
                wait_out(ahead_slot)

            issue_gather(c + NBUF - 1, ahead_slot)

        wait_gather(slot)
        block = buf[slot]
        m = jnp.max(block, axis=1, keepdims=True)
        e = jnp.exp(block - m)
        s = jnp.sum(e, axis=1, keepdims=True)
        lse = (m + jnp.log(s)).reshape(CH)

        base = c * CH
        tvec = jnp.stack([t_sref[base + j] for j in range(CH)])
        lane = jax.lax.broadcasted_iota(jnp.int32, (CH, VOCAB), 1)
        tl = jnp.sum(jnp.where(lane == tvec[:, None], block, 0.0), axis=1)

        issue_out(c, slot)
        return acc + jnp.sum(lse - tl)

    total = jax.lax.fori_loop(0, nchunks, step, jnp.float32(0.0))
    for s0 in range(NBUF):
        wait_out(s0)
    loss_ref[0, 0] = total / N


def kernel(x, targets, embed_weight):
    B, T = x.shape
    N = B * T
    x_flat = x.reshape(N).astype(jnp.int32)
    t_flat = targets.reshape(N).astype(jnp.int32)

    grid_spec = pltpu.PrefetchScalarGridSpec(
        num_scalar_prefetch=2,
        grid=(1,),
        in_specs=[pl.BlockSpec(memory_space=pl.ANY)],
        out_specs=[
            pl.BlockSpec(memory_space=pl.ANY),
            pl.BlockSpec(memory_space=pltpu.SMEM),
        ],
        scratch_shapes=[
            pltpu.VMEM((NBUF, CH, VOCAB), jnp.float32),
            pltpu.SemaphoreType.DMA((NBUF,)),
            pltpu.SemaphoreType.DMA((NBUF,)),
        ],
    )

    logits, loss = pl.pallas_call(
        _body,
        grid_spec=grid_spec,
        out_shape=[
            jax.ShapeDtypeStruct((N, VOCAB), jnp.float32),
            jax.ShapeDtypeStruct((1, 1), jnp.float32),
        ],
    )(x_flat, t_flat, embed_weight)

    return (logits, loss[0, 0])
